# Initial kernel scaffold; baseline (speedup 1.0000x reference)
import functools
import math

import jax
import jax.numpy as jnp
from jax import lax
from jax.experimental import pallas as pl
from jax.experimental.pallas import tpu as pltpu
from jax.experimental.pallas import tpu_sc as plsc

L = 16
NC = 2
NS = 16
K = 128


def _mm_body(x_ref, w_ref, b_ref, o_ref):
    o_ref[...] = lax.dot_general(
        x_ref[...], w_ref[...], (((1,), (1,)), ((), ())),
        preferred_element_type=jnp.float32) + b_ref[...]


def _linear_halves(x, W, b2, n_blk):
    n, d_in = x.shape
    d_out = W.shape[0]
    h = d_out // 2
    nb = n // n_blk
    return pl.pallas_call(
        _mm_body,
        grid=(nb, 2),
        in_specs=[
            pl.BlockSpec((n_blk, d_in), lambda i, c: (i, 0)),
            pl.BlockSpec((h, d_in), lambda i, c: (c, 0)),
            pl.BlockSpec((1, h), lambda i, c: (c, 0)),
        ],
        out_specs=pl.BlockSpec((n_blk, h), lambda i, c: (c * nb + i, 0)),
        out_shape=jax.ShapeDtypeStruct((2 * n, h), jnp.float32),
    )(x, W, b2)


def _make_scatter(n, h, nch, rpw):
    rpt = n // NS
    nzc = rpt // rpw
    mesh = plsc.VectorSubcoreMesh(core_axis_name="c", subcore_axis_name="s")

    @functools.partial(
        pl.kernel,
        out_type=jax.ShapeDtypeStruct((n, 2 * h), jnp.float32),
        mesh=mesh,
        scratch_types=[
            pltpu.VMEM_SHARED((n, h), jnp.float32),
            pltpu.VMEM((nch, K), jnp.int32),
            pltpu.VMEM((nch, K), jnp.int32),
            pltpu.VMEM((nch, K), jnp.float32),
            pltpu.VMEM((K, h), jnp.float32),
            pltpu.VMEM((rpw, h), jnp.float32),
            pltpu.SemaphoreType.DMA,
        ],
    )
    def scatter(sup_hbm, cols_hbm, rows_hbm, w_hbm, out_hbm,
                acc, col_v, row_v, w_v, msg, stg, sem):
        c = lax.axis_index("c")
        s = lax.axis_index("s")

        pltpu.sync_copy(cols_hbm.at[c, s], col_v)
        pltpu.sync_copy(rows_hbm.at[s], row_v)
        pltpu.sync_copy(w_hbm.at[s], w_v)

        def zrow(i, _):
            for u in range(h // L):
                stg[i, pl.ds(u * L, L)] = jnp.zeros((L,), jnp.float32)
            return 0
        lax.fori_loop(0, rpw, zrow, 0)
        base = s * rpt
        for z in range(nzc):
            pltpu.sync_copy(stg, acc.at[pl.ds(base + z * rpw, rpw)])
        plsc.subcore_barrier()

        def edge_chunk(j, _):
            pltpu.async_copy(sup_hbm.at[col_v.at[j]], msg, sem).wait()

            def scale(k, _):
                w = w_v[j, k]
                for u in range(h // L):
                    sl = pl.ds(u * L, L)
                    msg[k, sl] = w * msg[k, sl]
                return 0
            lax.fori_loop(0, K, scale, 0)
            pltpu.sync_copy(msg, acc.at[row_v.at[j]], add=True)
            return 0
        lax.fori_loop(0, nch, edge_chunk, 0)
        plsc.subcore_barrier()

        def wb(z, _):
            r0 = base + z * rpw
            pltpu.sync_copy(acc.at[pl.ds(r0, rpw)], stg)

            def relu_row(i, _):
                for u in range(h // L):
                    sl = pl.ds(u * L, L)
                    stg[i, sl] = jnp.maximum(stg[i, sl], 0.0)
                return 0
            lax.fori_loop(0, rpw, relu_row, 0)
            pltpu.sync_copy(stg, out_hbm.at[pl.ds(r0, rpw), pl.ds(c * h, h)])
            return 0
        lax.fori_loop(0, nzc, wb, 0)

    return scatter


def kernel(x, edge_index, edge_weight, W, b):
    n, d_in = x.shape
    d_out = W.shape[0]
    h = d_out // 2
    e = edge_index.shape[1]

    ept = math.ceil(e / (NS * K)) * K
    nch = ept // K
    e_pad = NS * ept
    pad = e_pad - e

    row = jnp.pad(edge_index[0], (0, pad))
    col = jnp.pad(edge_index[1], (0, pad))
    w = jnp.pad(edge_weight, (0, pad))

    cols2 = jnp.stack([col, col + n]).reshape(NC, NS, nch, K)
    rows3 = row.reshape(NS, nch, K)
    w3 = w.reshape(NS, nch, K)

    support = _linear_halves(x, W, b.reshape(2, h), n_blk=1000)
    scatter = _make_scatter(n, h, nch, rpw=125)
    return scatter(support, cols2, rows3, w3)


# speedup vs baseline: 4.0216x; 4.0216x over previous
import functools
import math

import jax
import jax.numpy as jnp
from jax import lax
from jax.experimental import pallas as pl
from jax.experimental.pallas import tpu as pltpu
from jax.experimental.pallas import tpu_sc as plsc

L = 16
NC = 2
NS = 16
K = 128


def _mm_body(x_ref, w_ref, b_ref, o_ref):
    o_ref[...] = lax.dot_general(
        x_ref[...], w_ref[...], (((1,), (1,)), ((), ())),
        preferred_element_type=jnp.float32) + b_ref[0]


def _linear_halves(x, W, b2, n_blk):
    n, d_in = x.shape
    d_out = W.shape[0]
    h = d_out // 2
    nb = n // n_blk
    return pl.pallas_call(
        _mm_body,
        grid=(nb, 2),
        in_specs=[
            pl.BlockSpec((n_blk, d_in), lambda i, c: (i, 0)),
            pl.BlockSpec((h, d_in), lambda i, c: (c, 0)),
            pl.BlockSpec((1, 1, h), lambda i, c: (c, 0, 0)),
        ],
        out_specs=pl.BlockSpec((n_blk, h), lambda i, c: (c * nb + i, 0)),
        out_shape=jax.ShapeDtypeStruct((2 * n, h), jnp.float32),
    )(x, W, b2)


def _make_scatter(n, h, nch, rpw):
    nq = n // rpw
    zpt = math.ceil(nq / NS)
    mesh = plsc.VectorSubcoreMesh(core_axis_name="c", subcore_axis_name="s",
                                  num_cores=NC, num_subcores=NS)

    @functools.partial(
        pl.kernel,
        out_type=jax.ShapeDtypeStruct((n, 2 * h), jnp.float32),
        mesh=mesh,
        scratch_types=[
            pltpu.VMEM_SHARED((n, h), jnp.float32),
            pltpu.VMEM((nch, K), jnp.int32),
            pltpu.VMEM((nch, K), jnp.int32),
            pltpu.VMEM((nch, K), jnp.float32),
            pltpu.VMEM((K, h), jnp.float32),
            pltpu.SemaphoreType.DMA,
        ],
    )
    def scatter(sup_hbm, cols_hbm, rows_hbm, w_hbm, out_hbm,
                acc, col_v, row_v, w_v, msg, sem):
        stg = msg.at[pl.ds(0, rpw)]
        c = lax.axis_index("c")
        s = lax.axis_index("s")

        pltpu.sync_copy(cols_hbm.at[c, s], col_v)
        pltpu.sync_copy(rows_hbm.at[s], row_v)
        pltpu.sync_copy(w_hbm.at[s], w_v)

        def zrow(i, _):
            for u in range(h // L):
                stg[i, pl.ds(u * L, L)] = jnp.zeros((L,), jnp.float32)
            return 0
        lax.fori_loop(0, rpw, zrow, 0)
        for z in range(zpt):
            q = z * NS + s

            @pl.when(q < nq)
            def _():
                pltpu.sync_copy(stg, acc.at[pl.ds(q * rpw, rpw)])
        plsc.subcore_barrier()

        def edge_chunk(j, _):
            pltpu.async_copy(sup_hbm.at[col_v.at[j]], msg, sem).wait()

            def scale16(g, _):
                wv = w_v[j, pl.ds(g * L, L)]
                for kk in range(L):
                    w = wv[kk]
                    k = g * L + kk
                    for u in range(h // L):
                        sl = pl.ds(u * L, L)
                        msg[k, sl] = w * msg[k, sl]
                return 0
            lax.fori_loop(0, K // L, scale16, 0)
            pltpu.sync_copy(msg, acc.at[row_v.at[j]], add=True)
            return 0
        lax.fori_loop(0, nch, edge_chunk, 0)
        plsc.subcore_barrier()

        for z in range(zpt):
            q = z * NS + s

            @pl.when(q < nq)
            def _():
                r0 = q * rpw
                pltpu.sync_copy(acc.at[pl.ds(r0, rpw)], stg)

                def relu_row(i, _):
                    for u in range(h // L):
                        sl = pl.ds(u * L, L)
                        stg[i, sl] = jnp.maximum(stg[i, sl], 0.0)
                    return 0
                lax.fori_loop(0, rpw, relu_row, 0)
                pltpu.sync_copy(stg, out_hbm.at[pl.ds(r0, rpw),
                                                pl.ds(c * h, h)])

    return scatter


def kernel(x, edge_index, edge_weight, W, b):
    n, d_in = x.shape
    d_out = W.shape[0]
    h = d_out // 2
    e = edge_index.shape[1]

    ept = math.ceil(e / (NS * K)) * K
    nch = ept // K
    e_pad = NS * ept
    pad = e_pad - e

    row = jnp.pad(edge_index[0], (0, pad))
    col = jnp.pad(edge_index[1], (0, pad))
    w = jnp.pad(edge_weight, (0, pad))

    cols2 = jnp.stack([col, col + n]).reshape(NC, NS, nch, K)
    rows3 = row.reshape(NS, nch, K)
    w3 = w.reshape(NS, nch, K)

    support = _linear_halves(x, W, b.reshape(2, 1, h), n_blk=1000)
    scatter = _make_scatter(n, h, nch, rpw=80)
    return scatter(support, cols2, rows3, w3)
